# baseline (speedup 1.0000x reference)
import functools

import jax
import jax.numpy as jnp
from jax import lax
from jax.experimental import pallas as pl
from jax.experimental.pallas import tpu as pltpu
from jax.experimental.pallas import tpu_sc as plsc

_NC = 2
_NS = 16
_NW = _NC * _NS
_G = 128
_NBUF = 4


@functools.lru_cache(maxsize=None)
def _gather_call(ngroups: int, D: int):
    mesh = plsc.VectorSubcoreMesh(core_axis_name="c", subcore_axis_name="s")

    @functools.partial(
        pl.kernel,
        mesh=mesh,
        out_type=jax.ShapeDtypeStruct((_NW, ngroups, _G, D), jnp.float32),
        scratch_types=(
            [pltpu.VMEM((ngroups, _G), jnp.int32)]
            + [pltpu.VMEM((_G, D), jnp.float32) for _ in range(_NBUF)]
            + [pltpu.SemaphoreType.DMA for _ in range(2 * _NBUF)]
        ),
        compiler_params=pltpu.CompilerParams(use_tc_tiling_on_sc=False),
    )
    def run(table, ids, out, idx_v, *rest):
        bufs = rest[:_NBUF]
        gsem = rest[_NBUF:2 * _NBUF]
        osem = rest[2 * _NBUF:]
        wid = lax.axis_index("s") * _NC + lax.axis_index("c")

        pltpu.sync_copy(ids.at[wid], idx_v)

        for b in range(_NBUF):
            pltpu.async_copy(table.at[idx_v.at[b]], bufs[b], gsem[b])

        nj = ngroups // _NBUF

        def step(j, carry):
            for b in range(_NBUF):
                g = j * _NBUF + b
                pltpu.make_async_copy(table.at[idx_v.at[g]], bufs[b],
                                      gsem[b]).wait()
                pltpu.async_copy(bufs[b], out.at[wid, g], osem[b])

                @pl.when(j < nj - 1)
                def _():
                    pltpu.make_async_copy(bufs[b], out.at[wid, g],
                                          osem[b]).wait()
                    pltpu.async_copy(table.at[idx_v.at[g + _NBUF]], bufs[b],
                                     gsem[b])

            return carry

        lax.fori_loop(0, nj, step, 0)

        for b in range(_NBUF):
            g = (nj - 1) * _NBUF + b
            pltpu.make_async_copy(bufs[b], out.at[wid, g], osem[b]).wait()

    return run


def _tc_transpose_body(a_ref, o_ref):
    t = a_ref[0].T
    o_ref[0, :, 0] = t[:64].reshape(8, 8, 128)
    o_ref[1, :, 0] = t[64:].reshape(8, 8, 128)


@functools.lru_cache(maxsize=None)
def _tc_transpose_call(S: int, B: int):
    return pl.pallas_call(
        _tc_transpose_body,
        grid=(S // 2, B // 128),
        in_specs=[pl.BlockSpec((1, 128, 128), lambda j, bt: (j, bt, 0))],
        out_specs=pl.BlockSpec((2, 8, 1, 8, 128),
                               lambda j, bt: (j, 0, bt, 0, 0)),
        out_shape=jax.ShapeDtypeStruct((S, 8, B // 128, 8, 128), jnp.float32),
    )


def kernel(token_ids, W):
    B, S = token_ids.shape
    V, D = W.shape
    total = B * S
    assert total % (_NW * _G) == 0 and S % 2 == 0 and D == 64
    ngroups = total // (_NW * _G)

    ids_p = (token_ids.astype(jnp.int32)
             .reshape(B, S // 2, 2)
             .transpose(1, 0, 2)
             .reshape(_NW, ngroups, _G))
    rows = _gather_call(ngroups, D)(W, ids_p)
    pairs = rows.reshape(S // 2, B, 2 * D)
    L = _tc_transpose_call(S, B)(pairs)
    return jnp.transpose(L, (2, 4, 0, 1, 3)).reshape(B, S, D)


# speedup vs baseline: 2.0981x; 2.0981x over previous
import functools

import jax
import jax.numpy as jnp
from jax import lax
from jax.experimental import pallas as pl
from jax.experimental.pallas import tpu as pltpu
from jax.experimental.pallas import tpu_sc as plsc

_NC = 2
_NS = 16
_NW = _NC * _NS
_G = 128
_NBUF = 8


@functools.lru_cache(maxsize=None)
def _emb_call(ngroups: int, D: int):
    mesh = plsc.VectorSubcoreMesh(core_axis_name="c", subcore_axis_name="s")

    @functools.partial(
        pl.kernel,
        mesh=mesh,
        out_type=jax.ShapeDtypeStruct((_NW, ngroups, _G, D), jnp.float32),
        scratch_types=(
            [pltpu.VMEM((ngroups, _G), jnp.int32)]
            + [pltpu.VMEM((_G, D), jnp.float32) for _ in range(_NBUF)]
            + [pltpu.SemaphoreType.DMA for _ in range(2 * _NBUF)]
        ),
        compiler_params=pltpu.CompilerParams(use_tc_tiling_on_sc=False),
    )
    def run(table, ids, out, idx_v, *rest):
        bufs = rest[:_NBUF]
        gsem = rest[_NBUF:2 * _NBUF]
        osem = rest[2 * _NBUF:]
        wid = lax.axis_index("s") * _NC + lax.axis_index("c")

        pltpu.sync_copy(ids.at[wid], idx_v)

        for b in range(_NBUF):
            pltpu.async_copy(table.at[idx_v.at[b]], bufs[b], gsem[b])

        nj = ngroups // _NBUF

        def step(j, carry):
            for b in range(_NBUF):
                g = j * _NBUF + b
                pltpu.make_async_copy(table.at[idx_v.at[g]], bufs[b], gsem[b]).wait()
                pltpu.async_copy(bufs[b], out.at[wid, g], osem[b])

                @pl.when(j < nj - 1)
                def _():
                    pltpu.make_async_copy(bufs[b], out.at[wid, g], osem[b]).wait()
                    pltpu.async_copy(table.at[idx_v.at[g + _NBUF]], bufs[b], gsem[b])

            return carry

        lax.fori_loop(0, nj, step, 0)

        for b in range(_NBUF):
            g = (nj - 1) * _NBUF + b
            pltpu.make_async_copy(bufs[b], out.at[wid, g], osem[b]).wait()

    return run


def kernel(token_ids, W):
    B, S = token_ids.shape
    V, D = W.shape
    total = B * S
    assert total % (_NW * _G) == 0
    ngroups = total // (_NW * _G)
    ids = token_ids.reshape(_NW, ngroups, _G).astype(jnp.int32)
    out = _emb_call(ngroups, D)(W, ids)
    return out.reshape(B, S, D)
